# baseline (speedup 1.0000x reference)
import functools
import math

import jax
import jax.numpy as jnp
from jax import lax
from jax.experimental import pallas as pl
from jax.experimental.pallas import tpu as pltpu
from jax.experimental.pallas import tpu_sc as plsc

NC = 2
NS = 16
CH = 128
IB = 16
ZR = 16
CW = 128
BN_EPS = 1e-5


def _zero_rows(zbuf, width):
    zero16 = jnp.zeros((16,), jnp.float32)
    for i in range(ZR):
        for j in range(width // 16):
            zbuf[i, pl.ds(16 * j, 16)] = zero16


def _sc_body(width, nch0, nch1, nacc, p_hbm, src_hbm, dst_hbm, out_hbm,
             acc, srcv, dstv, g0, g1, zbuf, sem0, sem1):
    c = lax.axis_index("c")
    s = lax.axis_index("s")
    rows_per_sub = nacc // NS

    _zero_rows(zbuf, width)

    row0 = s * rows_per_sub
    for k in range(rows_per_sub // ZR):
        pltpu.sync_copy(zbuf, acc.at[pl.ds(row0 + k * ZR, ZR)])
    plsc.subcore_barrier()

    base = jnp.where(c == 0, s * nch0, NS * nch0 + s * nch1)
    my_nb = jnp.where(c == 0, nch0 // IB, nch1 // IB)
    bufs = (g0, g1)
    sems = (sem0, sem1)
    for b in range(max(nch0, nch1) // IB):
        if True:
            cs = base + b * IB
            pltpu.sync_copy(src_hbm.at[pl.ds(cs, IB)], srcv)
            pltpu.sync_copy(dst_hbm.at[pl.ds(cs, IB)], dstv)
            desc = {0: pltpu.async_copy(p_hbm.at[srcv.at[0]], bufs[0],
                                        sems[0])}
            for i in range(IB):
                desc[i % 2].wait()
                if i + 1 < IB:
                    desc[(i + 1) % 2] = pltpu.async_copy(
                        p_hbm.at[srcv.at[i + 1]], bufs[(i + 1) % 2],
                        sems[(i + 1) % 2])
                pltpu.sync_copy(bufs[i % 2], acc.at[dstv.at[i]], add=True)
    plsc.subcore_barrier()

    pltpu.sync_copy(acc.at[pl.ds(row0, rows_per_sub)],
                    out_hbm.at[c, pl.ds(row0, rows_per_sub)])


def _sc_cnt_body(nch0, nch1, nacc, dst_hbm, cnt_hbm, cacc, dstv, ones, zbuf):
    c = lax.axis_index("c")
    s = lax.axis_index("s")
    rows_per_sub = nacc // NS

    one16 = jnp.ones((16,), jnp.float32)
    _zero_rows(zbuf, CW)
    for i in range(CH):
        for j in range(CW // 16):
            ones[i, pl.ds(16 * j, 16)] = one16

    row0 = s * rows_per_sub
    for k in range(rows_per_sub // ZR):
        pltpu.sync_copy(zbuf, cacc.at[pl.ds(row0 + k * ZR, ZR)])
    plsc.subcore_barrier()

    base = jnp.where(c == 0, s * nch0, NS * nch0 + s * nch1)
    my_nb = jnp.where(c == 0, nch0 // IB, nch1 // IB)

    def step(j, carry):
        pltpu.sync_copy(ones, cacc.at[dstv.at[j]], add=True)
        return carry

    for b in range(max(nch0, nch1) // IB):
        @pl.when(b < my_nb)
        def _block():
            pltpu.sync_copy(dst_hbm.at[pl.ds(base + b * IB, IB)], dstv)
            lax.fori_loop(0, IB, step, 0)
    plsc.subcore_barrier()

    pltpu.sync_copy(cacc.at[pl.ds(row0, rows_per_sub)],
                    cnt_hbm.at[c, pl.ds(row0, rows_per_sub)])


def _sc_mesh():
    return plsc.VectorSubcoreMesh(core_axis_name="c", subcore_axis_name="s",
                                  num_cores=NC, num_subcores=NS)


@functools.lru_cache(maxsize=None)
def _make_sc_scatter(width, nch0, nch1, nacc):
    scratch = (
        pltpu.VMEM_SHARED((nacc, width), jnp.float32),
        pltpu.VMEM((IB, CH), jnp.int32),
        pltpu.VMEM((IB, CH), jnp.int32),
        pltpu.VMEM((CH, width), jnp.float32),
        pltpu.VMEM((CH, width), jnp.float32),
        pltpu.VMEM((ZR, width), jnp.float32),
        pltpu.SemaphoreType.DMA,
        pltpu.SemaphoreType.DMA,
    )
    body = functools.partial(_sc_body, width, nch0, nch1, nacc)
    return pl.kernel(body,
                     out_type=jax.ShapeDtypeStruct((NC, nacc, width),
                                                   jnp.float32),
                     mesh=_sc_mesh(), scratch_types=scratch)


@functools.lru_cache(maxsize=None)
def _make_sc_cnt(nch0, nch1, nacc):
    scratch = (
        pltpu.VMEM_SHARED((nacc, CW), jnp.float32),
        pltpu.VMEM((IB, CH), jnp.int32),
        pltpu.VMEM((CH, CW), jnp.float32),
        pltpu.VMEM((ZR, CW), jnp.float32),
    )
    body = functools.partial(_sc_cnt_body, nch0, nch1, nacc)
    return pl.kernel(body,
                     out_type=jax.ShapeDtypeStruct((NC, nacc, CW),
                                                   jnp.float32),
                     mesh=_sc_mesh(), scratch_types=scratch)


_DN = (((1,), (1,)), ((), ()))


def _mm_in_body(x_ref, wl_ref, wr_ref, bl_ref, p_ref, r_ref):
    h = x_ref[...]
    p_ref[...] = lax.dot_general(h, wl_ref[...], _DN,
                                 preferred_element_type=jnp.float32)
    r_ref[...] = lax.dot_general(h, wr_ref[...], _DN,
                                 preferred_element_type=jnp.float32) + bl_ref[...]


def _combine_mm_body(s0_ref, s1_ref, r_ref, c0_ref, c1_ref, g_ref, be_ref,
                     wl_ref, wr_ref, bl_ref, p_ref, rn_ref):
    cnt = c0_ref[...][:, :1] + c1_ref[...][:, :1]
    rc = 1.0 / jnp.maximum(cnt, 1.0)
    h = (s0_ref[...] + s1_ref[...]) * rc + r_ref[...]
    scale = g_ref[...] * (1.0 / math.sqrt(1.0 + BN_EPS))
    h = jnp.maximum(h * scale + be_ref[...], 0.0)
    p_ref[...] = lax.dot_general(h, wl_ref[...], _DN,
                                 preferred_element_type=jnp.float32)
    rn_ref[...] = lax.dot_general(h, wr_ref[...], _DN,
                                  preferred_element_type=jnp.float32) + bl_ref[...]


def _final_body(s0_ref, s1_ref, r_ref, c0_ref, c1_ref, o_ref):
    cnt = c0_ref[...][:, :1] + c1_ref[...][:, :1]
    rc = 1.0 / jnp.maximum(cnt, 1.0)
    r = r_ref[...]
    agg = (s0_ref[...] + s1_ref[...]) * rc
    o_ref[...] = agg[:, :r.shape[1]] + r


def _row_spec(nb, w):
    return pl.BlockSpec((nb, w), lambda i: (i, 0))


def _full_spec(shape):
    return pl.BlockSpec(shape, lambda i: tuple(0 for _ in shape))


def kernel(x, edge_index, Wl0, bl0, Wr0, Wl1, bl1, Wr1, Wl2, bl2, Wr2,
           g0, be0, g1, be1):
    n, d = x.shape
    e = edge_index.shape[1]
    h_dim = Wl0.shape[0]
    o_dim = Wl2.shape[0]

    tot = -(-e // CH)
    f0 = 0.5
    nch0 = max(IB, int(round(tot * f0 / (NS * IB))) * IB)
    nch1 = max(IB, -(-(tot - NS * nch0) // (NS * IB)) * IB)
    cap = NS * (nch0 + nch1)
    epad = cap * CH
    nacc = -(-(n + 1) // (NS * ZR)) * (NS * ZR)

    src = edge_index[0]
    dst = edge_index[1]
    pad = epad - e
    pad_dst = n + jnp.arange(pad, dtype=jnp.int32) % jnp.int32(nacc - n)
    src_r = jnp.concatenate(
        [src, jnp.zeros((pad,), jnp.int32)]).reshape(cap, CH)
    dst_r = jnp.concatenate([dst, pad_dst]).reshape(cap, CH)

    nb = 1000
    grid = (n // nb,)

    bl0_2, bl1_2, bl2_2 = bl0[None], bl1[None], bl2[None]
    g0_2, be0_2 = g0[None], be0[None]
    g1_2, be1_2 = g1[None], be1[None]

    p0, r0 = pl.pallas_call(
        _mm_in_body,
        grid=grid,
        in_specs=[_row_spec(nb, d), _full_spec((h_dim, d)),
                  _full_spec((h_dim, d)), _full_spec((1, h_dim))],
        out_specs=[_row_spec(nb, h_dim), _row_spec(nb, h_dim)],
        out_shape=[jax.ShapeDtypeStruct((n, h_dim), jnp.float32),
                   jax.ShapeDtypeStruct((n, h_dim), jnp.float32)],
    )(x, Wl0, Wr0, bl0_2)

    cp = _make_sc_cnt(nch0, nch1, nacc)(dst_r)
    s0p = _make_sc_scatter(h_dim, nch0, nch1, nacc)(p0, src_r, dst_r)
    s00, s01 = s0p[0, :n], s0p[1, :n]
    c0, c1 = cp[0, :n], cp[1, :n]

    p1, r1 = pl.pallas_call(
        _combine_mm_body,
        grid=grid,
        in_specs=[_row_spec(nb, h_dim), _row_spec(nb, h_dim),
                  _row_spec(nb, h_dim), _row_spec(nb, CW), _row_spec(nb, CW),
                  _full_spec((1, h_dim)), _full_spec((1, h_dim)),
                  _full_spec((h_dim, h_dim)), _full_spec((h_dim, h_dim)),
                  _full_spec((1, h_dim))],
        out_specs=[_row_spec(nb, h_dim), _row_spec(nb, h_dim)],
        out_shape=[jax.ShapeDtypeStruct((n, h_dim), jnp.float32),
                   jax.ShapeDtypeStruct((n, h_dim), jnp.float32)],
    )(s00, s01, r0, c0, c1, g0_2, be0_2, Wl1, Wr1, bl1_2)

    s1p = _make_sc_scatter(h_dim, nch0, nch1, nacc)(p1, src_r, dst_r)

    wl2p = jnp.concatenate(
        [Wl2, jnp.zeros((h_dim - o_dim, h_dim), jnp.float32)], axis=0)
    p2, r2 = pl.pallas_call(
        _combine_mm_body,
        grid=grid,
        in_specs=[_row_spec(nb, h_dim), _row_spec(nb, h_dim),
                  _row_spec(nb, h_dim), _row_spec(nb, CW), _row_spec(nb, CW),
                  _full_spec((1, h_dim)), _full_spec((1, h_dim)),
                  _full_spec((h_dim, h_dim)), _full_spec((o_dim, h_dim)),
                  _full_spec((1, o_dim))],
        out_specs=[_row_spec(nb, h_dim), _row_spec(nb, o_dim)],
        out_shape=[jax.ShapeDtypeStruct((n, h_dim), jnp.float32),
                   jax.ShapeDtypeStruct((n, o_dim), jnp.float32)],
    )(s1p[0, :n], s1p[1, :n], r1, c0, c1, g1_2, be1_2, wl2p, Wr2, bl2_2)

    s2p = _make_sc_scatter(h_dim, nch0, nch1, nacc)(p2, src_r, dst_r)

    out = pl.pallas_call(
        _final_body,
        grid=grid,
        in_specs=[_row_spec(nb, h_dim), _row_spec(nb, h_dim),
                  _row_spec(nb, o_dim), _row_spec(nb, CW), _row_spec(nb, CW)],
        out_specs=_row_spec(nb, o_dim),
        out_shape=jax.ShapeDtypeStruct((n, o_dim), jnp.float32),
    )(s2p[0, :n], s2p[1, :n], r2, c0, c1)

    return out


# speedup vs baseline: 1.1090x; 1.1090x over previous
import functools
import math

import jax
import jax.numpy as jnp
from jax import lax
from jax.experimental import pallas as pl
from jax.experimental.pallas import tpu as pltpu
from jax.experimental.pallas import tpu_sc as plsc

NC = 2
NS = 16
CH = 128
IB = 16
ZR = 16
CW = 128
BN_EPS = 1e-5


def _zero_rows(zbuf, width):
    zero16 = jnp.zeros((16,), jnp.float32)
    for i in range(ZR):
        for j in range(width // 16):
            zbuf[i, pl.ds(16 * j, 16)] = zero16


def _sc_body(width, nch0, nch1, nacc, p_hbm, src0_hbm, dst0_hbm, src1_hbm,
             dst1_hbm, out_hbm, acc, srcv, dstv, g0, g1, zbuf, sem0, sem1):
    c = lax.axis_index("c")
    s = lax.axis_index("s")
    rows_per_sub = nacc // NS

    _zero_rows(zbuf, width)

    row0 = s * rows_per_sub
    for k in range(rows_per_sub // ZR):
        pltpu.sync_copy(zbuf, acc.at[pl.ds(row0 + k * ZR, ZR)])
    plsc.subcore_barrier()

    bufs = (g0, g1)
    sems = (sem0, sem1)

    def run_blocks(src_hbm, dst_hbm, nch):
        for b in range(nch // IB):
            cs = b * IB
            pltpu.sync_copy(src_hbm.at[s, pl.ds(cs, IB)], srcv)
            pltpu.sync_copy(dst_hbm.at[s, pl.ds(cs, IB)], dstv)
            desc = {0: pltpu.async_copy(p_hbm.at[srcv.at[0]], bufs[0],
                                        sems[0])}
            for i in range(IB):
                desc[i % 2].wait()
                if i + 1 < IB:
                    desc[(i + 1) % 2] = pltpu.async_copy(
                        p_hbm.at[srcv.at[i + 1]], bufs[(i + 1) % 2],
                        sems[(i + 1) % 2])
                pltpu.sync_copy(bufs[i % 2], acc.at[dstv.at[i]], add=True)

    @pl.when(c == 0)
    def _core0():
        run_blocks(src0_hbm, dst0_hbm, nch0)

    @pl.when(c == 1)
    def _core1():
        run_blocks(src1_hbm, dst1_hbm, nch1)

    plsc.subcore_barrier()

    pltpu.sync_copy(acc.at[pl.ds(row0, rows_per_sub)],
                    out_hbm.at[c, pl.ds(row0, rows_per_sub)])


def _sc_cnt_body(nch0, nch1, nacc, dst0_hbm, dst1_hbm, cnt_hbm, cacc, dstv,
                 ones, zbuf):
    c = lax.axis_index("c")
    s = lax.axis_index("s")
    rows_per_sub = nacc // NS

    one16 = jnp.ones((16,), jnp.float32)
    _zero_rows(zbuf, CW)
    for i in range(CH):
        for j in range(CW // 16):
            ones[i, pl.ds(16 * j, 16)] = one16

    row0 = s * rows_per_sub
    for k in range(rows_per_sub // ZR):
        pltpu.sync_copy(zbuf, cacc.at[pl.ds(row0 + k * ZR, ZR)])
    plsc.subcore_barrier()

    def step(j, carry):
        pltpu.sync_copy(ones, cacc.at[dstv.at[j]], add=True)
        return carry

    def run_blocks(dst_hbm, nch):
        for b in range(nch // IB):
            pltpu.sync_copy(dst_hbm.at[s, pl.ds(b * IB, IB)], dstv)
            lax.fori_loop(0, IB, step, 0)

    @pl.when(c == 0)
    def _core0():
        run_blocks(dst0_hbm, nch0)

    @pl.when(c == 1)
    def _core1():
        run_blocks(dst1_hbm, nch1)

    plsc.subcore_barrier()

    pltpu.sync_copy(cacc.at[pl.ds(row0, rows_per_sub)],
                    cnt_hbm.at[c, pl.ds(row0, rows_per_sub)])


def _sc_mesh():
    return plsc.VectorSubcoreMesh(core_axis_name="c", subcore_axis_name="s",
                                  num_cores=NC, num_subcores=NS)


@functools.lru_cache(maxsize=None)
def _make_sc_scatter(width, nch0, nch1, nacc):
    scratch = (
        pltpu.VMEM_SHARED((nacc, width), jnp.float32),
        pltpu.VMEM((IB, CH), jnp.int32),
        pltpu.VMEM((IB, CH), jnp.int32),
        pltpu.VMEM((CH, width), jnp.float32),
        pltpu.VMEM((CH, width), jnp.float32),
        pltpu.VMEM((ZR, width), jnp.float32),
        pltpu.SemaphoreType.DMA,
        pltpu.SemaphoreType.DMA,
    )
    body = functools.partial(_sc_body, width, nch0, nch1, nacc)
    return pl.kernel(body,
                     out_type=jax.ShapeDtypeStruct((NC, nacc, width),
                                                   jnp.float32),
                     mesh=_sc_mesh(), scratch_types=scratch)


@functools.lru_cache(maxsize=None)
def _make_sc_cnt(nch0, nch1, nacc):
    scratch = (
        pltpu.VMEM_SHARED((nacc, CW), jnp.float32),
        pltpu.VMEM((IB, CH), jnp.int32),
        pltpu.VMEM((CH, CW), jnp.float32),
        pltpu.VMEM((ZR, CW), jnp.float32),
    )
    body = functools.partial(_sc_cnt_body, nch0, nch1, nacc)
    return pl.kernel(body,
                     out_type=jax.ShapeDtypeStruct((NC, nacc, CW),
                                                   jnp.float32),
                     mesh=_sc_mesh(), scratch_types=scratch)


_DN = (((1,), (1,)), ((), ()))


def _mm_in_body(x_ref, wl_ref, wr_ref, bl_ref, p_ref, r_ref):
    h = x_ref[...]
    p_ref[...] = lax.dot_general(h, wl_ref[...], _DN,
                                 preferred_element_type=jnp.float32)
    r_ref[...] = lax.dot_general(h, wr_ref[...], _DN,
                                 preferred_element_type=jnp.float32) + bl_ref[...]


def _combine_mm_body(s0_ref, s1_ref, r_ref, c0_ref, c1_ref, g_ref, be_ref,
                     wl_ref, wr_ref, bl_ref, p_ref, rn_ref):
    cnt = c0_ref[...][:, :1] + c1_ref[...][:, :1]
    rc = 1.0 / jnp.maximum(cnt, 1.0)
    h = (s0_ref[...] + s1_ref[...]) * rc + r_ref[...]
    scale = g_ref[...] * (1.0 / math.sqrt(1.0 + BN_EPS))
    h = jnp.maximum(h * scale + be_ref[...], 0.0)
    p_ref[...] = lax.dot_general(h, wl_ref[...], _DN,
                                 preferred_element_type=jnp.float32)
    rn_ref[...] = lax.dot_general(h, wr_ref[...], _DN,
                                  preferred_element_type=jnp.float32) + bl_ref[...]


def _final_body(s0_ref, s1_ref, r_ref, c0_ref, c1_ref, o_ref):
    cnt = c0_ref[...][:, :1] + c1_ref[...][:, :1]
    rc = 1.0 / jnp.maximum(cnt, 1.0)
    r = r_ref[...]
    agg = (s0_ref[...] + s1_ref[...]) * rc
    o_ref[...] = agg[:, :r.shape[1]] + r


def _row_spec(nb, w):
    return pl.BlockSpec((nb, w), lambda i: (i, 0))


def _full_spec(shape):
    return pl.BlockSpec(shape, lambda i: tuple(0 for _ in shape))


def kernel(x, edge_index, Wl0, bl0, Wr0, Wl1, bl1, Wr1, Wl2, bl2, Wr2,
           g0, be0, g1, be1):
    n, d = x.shape
    e = edge_index.shape[1]
    h_dim = Wl0.shape[0]
    o_dim = Wl2.shape[0]

    tot = -(-e // CH)
    f0 = 0.6
    nch0 = max(IB, int(round(tot * f0 / (NS * IB))) * IB)
    nch1 = max(IB, -(-(tot - NS * nch0) // (NS * IB)) * IB)
    cap = NS * (nch0 + nch1)
    epad = cap * CH
    nacc = -(-(n + 1) // (NS * ZR)) * (NS * ZR)

    src = edge_index[0]
    dst = edge_index[1]
    pad = epad - e
    pad_dst = n + jnp.arange(pad, dtype=jnp.int32) % jnp.int32(nacc - n)
    src_p = jnp.concatenate([src, jnp.zeros((pad,), jnp.int32)])
    dst_p = jnp.concatenate([dst, pad_dst])
    cut = NS * nch0 * CH
    src0_r = src_p[:cut].reshape(NS, nch0, CH)
    dst0_r = dst_p[:cut].reshape(NS, nch0, CH)
    src1_r = src_p[cut:].reshape(NS, nch1, CH)
    dst1_r = dst_p[cut:].reshape(NS, nch1, CH)

    nb = 1000
    grid = (n // nb,)

    bl0_2, bl1_2, bl2_2 = bl0[None], bl1[None], bl2[None]
    g0_2, be0_2 = g0[None], be0[None]
    g1_2, be1_2 = g1[None], be1[None]

    p0, r0 = pl.pallas_call(
        _mm_in_body,
        grid=grid,
        in_specs=[_row_spec(nb, d), _full_spec((h_dim, d)),
                  _full_spec((h_dim, d)), _full_spec((1, h_dim))],
        out_specs=[_row_spec(nb, h_dim), _row_spec(nb, h_dim)],
        out_shape=[jax.ShapeDtypeStruct((n, h_dim), jnp.float32),
                   jax.ShapeDtypeStruct((n, h_dim), jnp.float32)],
    )(x, Wl0, Wr0, bl0_2)

    cp = _make_sc_cnt(nch0, nch1, nacc)(dst0_r, dst1_r)
    s0p = _make_sc_scatter(h_dim, nch0, nch1, nacc)(p0, src0_r, dst0_r, src1_r, dst1_r)
    s00, s01 = s0p[0, :n], s0p[1, :n]
    c0, c1 = cp[0, :n], cp[1, :n]

    p1, r1 = pl.pallas_call(
        _combine_mm_body,
        grid=grid,
        in_specs=[_row_spec(nb, h_dim), _row_spec(nb, h_dim),
                  _row_spec(nb, h_dim), _row_spec(nb, CW), _row_spec(nb, CW),
                  _full_spec((1, h_dim)), _full_spec((1, h_dim)),
                  _full_spec((h_dim, h_dim)), _full_spec((h_dim, h_dim)),
                  _full_spec((1, h_dim))],
        out_specs=[_row_spec(nb, h_dim), _row_spec(nb, h_dim)],
        out_shape=[jax.ShapeDtypeStruct((n, h_dim), jnp.float32),
                   jax.ShapeDtypeStruct((n, h_dim), jnp.float32)],
    )(s00, s01, r0, c0, c1, g0_2, be0_2, Wl1, Wr1, bl1_2)

    s1p = _make_sc_scatter(h_dim, nch0, nch1, nacc)(p1, src0_r, dst0_r, src1_r, dst1_r)

    wl2p = jnp.concatenate(
        [Wl2, jnp.zeros((h_dim - o_dim, h_dim), jnp.float32)], axis=0)
    p2, r2 = pl.pallas_call(
        _combine_mm_body,
        grid=grid,
        in_specs=[_row_spec(nb, h_dim), _row_spec(nb, h_dim),
                  _row_spec(nb, h_dim), _row_spec(nb, CW), _row_spec(nb, CW),
                  _full_spec((1, h_dim)), _full_spec((1, h_dim)),
                  _full_spec((h_dim, h_dim)), _full_spec((o_dim, h_dim)),
                  _full_spec((1, o_dim))],
        out_specs=[_row_spec(nb, h_dim), _row_spec(nb, o_dim)],
        out_shape=[jax.ShapeDtypeStruct((n, h_dim), jnp.float32),
                   jax.ShapeDtypeStruct((n, o_dim), jnp.float32)],
    )(s1p[0, :n], s1p[1, :n], r1, c0, c1, g1_2, be1_2, wl2p, Wr2, bl2_2)

    s2p = _make_sc_scatter(h_dim, nch0, nch1, nacc)(p2, src0_r, dst0_r, src1_r, dst1_r)

    out = pl.pallas_call(
        _final_body,
        grid=grid,
        in_specs=[_row_spec(nb, h_dim), _row_spec(nb, h_dim),
                  _row_spec(nb, o_dim), _row_spec(nb, CW), _row_spec(nb, CW)],
        out_specs=_row_spec(nb, o_dim),
        out_shape=jax.ShapeDtypeStruct((n, o_dim), jnp.float32),
    )(s2p[0, :n], s2p[1, :n], r2, c0, c1)

    return out


# speedup vs baseline: 1.8195x; 1.6406x over previous
import functools
import math

import jax
import jax.numpy as jnp
from jax import lax
from jax.experimental import pallas as pl
from jax.experimental.pallas import tpu as pltpu
from jax.experimental.pallas import tpu_sc as plsc

NC = 2
NS = 16
CH = 128
IB = 16
ZR = 16
CW = 128
BN_EPS = 1e-5


def _zero_rows(zbuf, width):
    zero16 = jnp.zeros((16,), jnp.float32)
    for i in range(ZR):
        for j in range(width // 16):
            zbuf[i, pl.ds(16 * j, 16)] = zero16


def _sc_body(width, nch, nacc, p_hbm, src_hbm, dst_hbm, out_hbm,
             acc, srcv, dstv, g0, g1, zbuf, sem0, sem1):
    c = lax.axis_index("c")
    s = lax.axis_index("s")
    wid = c * NS + s
    rows_per_sub = nacc // NS

    _zero_rows(zbuf, width)

    row0 = s * rows_per_sub
    for k in range(rows_per_sub // ZR):
        pltpu.sync_copy(zbuf, acc.at[pl.ds(row0 + k * ZR, ZR)])
    plsc.subcore_barrier()

    bufs = (g0, g1)
    sems = (sem0, sem1)
    for b in range(-(-nch // IB)):
        cs = b * IB
        ncur = min(IB, nch - cs)
        pltpu.sync_copy(src_hbm.at[wid, pl.ds(cs, ncur)],
                        srcv.at[pl.ds(0, ncur)])
        pltpu.sync_copy(dst_hbm.at[wid, pl.ds(cs, ncur)],
                        dstv.at[pl.ds(0, ncur)])
        desc = {0: pltpu.async_copy(p_hbm.at[srcv.at[0]], bufs[0], sems[0])}
        for i in range(ncur):
            desc[i % 2].wait()
            if i + 1 < ncur:
                desc[(i + 1) % 2] = pltpu.async_copy(
                    p_hbm.at[srcv.at[i + 1]], bufs[(i + 1) % 2],
                    sems[(i + 1) % 2])
            pltpu.sync_copy(bufs[i % 2], acc.at[dstv.at[i]], add=True)
    plsc.subcore_barrier()

    pltpu.sync_copy(acc.at[pl.ds(row0, rows_per_sub)],
                    out_hbm.at[c, pl.ds(row0, rows_per_sub)])


def _sc_cnt_body(nch, nacc, dst_hbm, cnt_hbm, cacc, dstv, ones, zbuf):
    c = lax.axis_index("c")
    s = lax.axis_index("s")
    wid = c * NS + s
    rows_per_sub = nacc // NS

    one16 = jnp.ones((16,), jnp.float32)
    _zero_rows(zbuf, CW)
    for i in range(CH):
        for j in range(CW // 16):
            ones[i, pl.ds(16 * j, 16)] = one16

    row0 = s * rows_per_sub
    for k in range(rows_per_sub // ZR):
        pltpu.sync_copy(zbuf, cacc.at[pl.ds(row0 + k * ZR, ZR)])
    plsc.subcore_barrier()

    def step(j, carry):
        pltpu.sync_copy(ones, cacc.at[dstv.at[j]], add=True)
        return carry

    for b in range(-(-nch // IB)):
        cs = b * IB
        ncur = min(IB, nch - cs)
        pltpu.sync_copy(dst_hbm.at[wid, pl.ds(cs, ncur)],
                        dstv.at[pl.ds(0, ncur)])
        lax.fori_loop(0, ncur, step, 0)
    plsc.subcore_barrier()

    pltpu.sync_copy(cacc.at[pl.ds(row0, rows_per_sub)],
                    cnt_hbm.at[c, pl.ds(row0, rows_per_sub)])


def _sc_mesh():
    return plsc.VectorSubcoreMesh(core_axis_name="c", subcore_axis_name="s",
                                  num_cores=NC, num_subcores=NS)


@functools.lru_cache(maxsize=None)
def _make_sc_scatter(width, nch, nacc):
    scratch = (
        pltpu.VMEM_SHARED((nacc, width), jnp.float32),
        pltpu.VMEM((IB, CH), jnp.int32),
        pltpu.VMEM((IB, CH), jnp.int32),
        pltpu.VMEM((CH, width), jnp.float32),
        pltpu.VMEM((CH, width), jnp.float32),
        pltpu.VMEM((ZR, width), jnp.float32),
        pltpu.SemaphoreType.DMA,
        pltpu.SemaphoreType.DMA,
    )
    body = functools.partial(_sc_body, width, nch, nacc)
    return pl.kernel(body,
                     out_type=jax.ShapeDtypeStruct((NC, nacc, width),
                                                   jnp.float32),
                     mesh=_sc_mesh(), scratch_types=scratch)


@functools.lru_cache(maxsize=None)
def _make_sc_cnt(nch, nacc):
    scratch = (
        pltpu.VMEM_SHARED((nacc, CW), jnp.float32),
        pltpu.VMEM((IB, CH), jnp.int32),
        pltpu.VMEM((CH, CW), jnp.float32),
        pltpu.VMEM((ZR, CW), jnp.float32),
    )
    body = functools.partial(_sc_cnt_body, nch, nacc)
    return pl.kernel(body,
                     out_type=jax.ShapeDtypeStruct((NC, nacc, CW),
                                                   jnp.float32),
                     mesh=_sc_mesh(), scratch_types=scratch)


_DN = (((1,), (1,)), ((), ()))


def _mm_in_body(x_ref, wl_ref, wr_ref, bl_ref, p_ref, r_ref):
    h = x_ref[...]
    p_ref[...] = lax.dot_general(h, wl_ref[...], _DN,
                                 preferred_element_type=jnp.float32)
    r_ref[...] = lax.dot_general(h, wr_ref[...], _DN,
                                 preferred_element_type=jnp.float32) + bl_ref[...]


def _combine_mm_body(s0_ref, s1_ref, r_ref, c0_ref, c1_ref, g_ref, be_ref,
                     wl_ref, wr_ref, bl_ref, p_ref, rn_ref):
    cnt = c0_ref[...][:, :1] + c1_ref[...][:, :1]
    rc = 1.0 / jnp.maximum(cnt, 1.0)
    h = (s0_ref[...] + s1_ref[...]) * rc + r_ref[...]
    scale = g_ref[...] * (1.0 / math.sqrt(1.0 + BN_EPS))
    h = jnp.maximum(h * scale + be_ref[...], 0.0)
    p_ref[...] = lax.dot_general(h, wl_ref[...], _DN,
                                 preferred_element_type=jnp.float32)
    rn_ref[...] = lax.dot_general(h, wr_ref[...], _DN,
                                  preferred_element_type=jnp.float32) + bl_ref[...]


def _final_body(s0_ref, s1_ref, r_ref, c0_ref, c1_ref, o_ref):
    cnt = c0_ref[...][:, :1] + c1_ref[...][:, :1]
    rc = 1.0 / jnp.maximum(cnt, 1.0)
    r = r_ref[...]
    agg = (s0_ref[...] + s1_ref[...]) * rc
    o_ref[...] = agg[:, :r.shape[1]] + r


def _row_spec(nb, w):
    return pl.BlockSpec((nb, w), lambda i: (i, 0))


def _full_spec(shape):
    return pl.BlockSpec(shape, lambda i: tuple(0 for _ in shape))


def kernel(x, edge_index, Wl0, bl0, Wr0, Wl1, bl1, Wr1, Wl2, bl2, Wr2,
           g0, be0, g1, be1):
    n, d = x.shape
    e = edge_index.shape[1]
    h_dim = Wl0.shape[0]
    o_dim = Wl2.shape[0]

    nch = -(-e // (NC * NS * CH))
    epad = NC * NS * CH * nch
    nacc = -(-(n + 1) // (NS * ZR)) * (NS * ZR)

    src = edge_index[0]
    dst = edge_index[1]
    pad = epad - e
    pad_dst = n + jnp.arange(pad, dtype=jnp.int32) % jnp.int32(nacc - n)
    src_r = jnp.concatenate(
        [src, jnp.zeros((pad,), jnp.int32)]).reshape(NC * NS, nch, CH)
    dst_r = jnp.concatenate([dst, pad_dst]).reshape(NC * NS, nch, CH)

    nb = 1000
    grid = (n // nb,)

    bl0_2, bl1_2, bl2_2 = bl0[None], bl1[None], bl2[None]
    g0_2, be0_2 = g0[None], be0[None]
    g1_2, be1_2 = g1[None], be1[None]

    p0, r0 = pl.pallas_call(
        _mm_in_body,
        grid=grid,
        in_specs=[_row_spec(nb, d), _full_spec((h_dim, d)),
                  _full_spec((h_dim, d)), _full_spec((1, h_dim))],
        out_specs=[_row_spec(nb, h_dim), _row_spec(nb, h_dim)],
        out_shape=[jax.ShapeDtypeStruct((n, h_dim), jnp.float32),
                   jax.ShapeDtypeStruct((n, h_dim), jnp.float32)],
    )(x, Wl0, Wr0, bl0_2)

    cp = _make_sc_cnt(nch, nacc)(dst_r)
    s0p = _make_sc_scatter(h_dim, nch, nacc)(p0, src_r, dst_r)
    s00, s01 = s0p[0, :n], s0p[1, :n]
    c0, c1 = cp[0, :n], cp[1, :n]

    p1, r1 = pl.pallas_call(
        _combine_mm_body,
        grid=grid,
        in_specs=[_row_spec(nb, h_dim), _row_spec(nb, h_dim),
                  _row_spec(nb, h_dim), _row_spec(nb, CW), _row_spec(nb, CW),
                  _full_spec((1, h_dim)), _full_spec((1, h_dim)),
                  _full_spec((h_dim, h_dim)), _full_spec((h_dim, h_dim)),
                  _full_spec((1, h_dim))],
        out_specs=[_row_spec(nb, h_dim), _row_spec(nb, h_dim)],
        out_shape=[jax.ShapeDtypeStruct((n, h_dim), jnp.float32),
                   jax.ShapeDtypeStruct((n, h_dim), jnp.float32)],
    )(s00, s01, r0, c0, c1, g0_2, be0_2, Wl1, Wr1, bl1_2)

    s1p = _make_sc_scatter(h_dim, nch, nacc)(p1, src_r, dst_r)

    wl2p = jnp.concatenate(
        [Wl2, jnp.zeros((h_dim - o_dim, h_dim), jnp.float32)], axis=0)
    p2, r2 = pl.pallas_call(
        _combine_mm_body,
        grid=grid,
        in_specs=[_row_spec(nb, h_dim), _row_spec(nb, h_dim),
                  _row_spec(nb, h_dim), _row_spec(nb, CW), _row_spec(nb, CW),
                  _full_spec((1, h_dim)), _full_spec((1, h_dim)),
                  _full_spec((h_dim, h_dim)), _full_spec((o_dim, h_dim)),
                  _full_spec((1, o_dim))],
        out_specs=[_row_spec(nb, h_dim), _row_spec(nb, o_dim)],
        out_shape=[jax.ShapeDtypeStruct((n, h_dim), jnp.float32),
                   jax.ShapeDtypeStruct((n, o_dim), jnp.float32)],
    )(s1p[0, :n], s1p[1, :n], r1, c0, c1, g1_2, be1_2, wl2p, Wr2, bl2_2)

    s2p = _make_sc_scatter(h_dim, nch, nacc)(p2, src_r, dst_r)

    out = pl.pallas_call(
        _final_body,
        grid=grid,
        in_specs=[_row_spec(nb, h_dim), _row_spec(nb, h_dim),
                  _row_spec(nb, o_dim), _row_spec(nb, CW), _row_spec(nb, CW)],
        out_specs=_row_spec(nb, o_dim),
        out_shape=jax.ShapeDtypeStruct((n, o_dim), jnp.float32),
    )(s2p[0, :n], s2p[1, :n], r2, c0, c1)

    return out


# speedup vs baseline: 1.8767x; 1.0315x over previous
import functools
import math

import jax
import jax.numpy as jnp
from jax import lax
from jax.experimental import pallas as pl
from jax.experimental.pallas import tpu as pltpu
from jax.experimental.pallas import tpu_sc as plsc

NC = 2
NS = 16
CH = 128
IB = 16
ZR = 16
CW = 128
BN_EPS = 1e-5


def _zero_rows(zbuf, width):
    zero16 = jnp.zeros((16,), jnp.float32)
    for i in range(ZR):
        for j in range(width // 16):
            zbuf[i, pl.ds(16 * j, 16)] = zero16


def _sc_body(width, nch, nacc, p_hbm, src_hbm, dst_hbm, out_hbm,
             acc, srcv, dstv, g0, g1, zbuf, sem0, sem1):
    c = lax.axis_index("c")
    s = lax.axis_index("s")
    wid = c * NS + s
    rows_per_sub = nacc // NS

    _zero_rows(zbuf, width)

    row0 = s * rows_per_sub
    for k in range(rows_per_sub // ZR):
        pltpu.sync_copy(zbuf, acc.at[pl.ds(row0 + k * ZR, ZR)])
    plsc.subcore_barrier()

    bufs = (g0, g1)
    sems = (sem0, sem1)
    for b in range(-(-nch // IB)):
        cs = b * IB
        ncur = min(IB, nch - cs)
        pltpu.sync_copy(src_hbm.at[wid, pl.ds(cs, ncur)],
                        srcv.at[pl.ds(0, ncur)])
        pltpu.sync_copy(dst_hbm.at[wid, pl.ds(cs, ncur)],
                        dstv.at[pl.ds(0, ncur)])
        desc = {0: pltpu.async_copy(p_hbm.at[srcv.at[0]], bufs[0], sems[0])}
        for i in range(ncur):
            desc[i % 2].wait()
            if i + 1 < ncur:
                desc[(i + 1) % 2] = pltpu.async_copy(
                    p_hbm.at[srcv.at[i + 1]], bufs[(i + 1) % 2],
                    sems[(i + 1) % 2])
            pltpu.sync_copy(bufs[i % 2], acc.at[dstv.at[i]], add=True)
    plsc.subcore_barrier()

    pltpu.sync_copy(acc.at[pl.ds(row0, rows_per_sub)],
                    out_hbm.at[c, pl.ds(row0, rows_per_sub)])


def _sc_cnt_body(nch, nacc, dst_hbm, cnt_hbm, cacc, dstv, ones, zbuf):
    c = lax.axis_index("c")
    s = lax.axis_index("s")
    wid = c * NS + s
    rows_per_sub = nacc // NS

    one16 = jnp.ones((16,), jnp.float32)
    _zero_rows(zbuf, CW)
    for i in range(CH):
        for j in range(CW // 16):
            ones[i, pl.ds(16 * j, 16)] = one16

    row0 = s * rows_per_sub
    for k in range(rows_per_sub // ZR):
        pltpu.sync_copy(zbuf, cacc.at[pl.ds(row0 + k * ZR, ZR)])
    plsc.subcore_barrier()

    def step(j, carry):
        pltpu.sync_copy(ones, cacc.at[dstv.at[j]], add=True)
        return carry

    for b in range(-(-nch // IB)):
        cs = b * IB
        ncur = min(IB, nch - cs)
        pltpu.sync_copy(dst_hbm.at[wid, pl.ds(cs, ncur)],
                        dstv.at[pl.ds(0, ncur)])
        lax.fori_loop(0, ncur, step, 0)
    plsc.subcore_barrier()

    pltpu.sync_copy(cacc.at[pl.ds(row0, rows_per_sub)],
                    cnt_hbm.at[c, pl.ds(row0, rows_per_sub)])


def _sc_mesh():
    return plsc.VectorSubcoreMesh(core_axis_name="c", subcore_axis_name="s",
                                  num_cores=NC, num_subcores=NS)


@functools.lru_cache(maxsize=None)
def _make_sc_scatter(width, nch, nacc):
    scratch = (
        pltpu.VMEM_SHARED((nacc, width), jnp.float32),
        pltpu.VMEM((IB, CH), jnp.int32),
        pltpu.VMEM((IB, CH), jnp.int32),
        pltpu.VMEM((CH, width), jnp.float32),
        pltpu.VMEM((CH, width), jnp.float32),
        pltpu.VMEM((ZR, width), jnp.float32),
        pltpu.SemaphoreType.DMA,
        pltpu.SemaphoreType.DMA,
    )
    body = functools.partial(_sc_body, width, nch, nacc)
    return pl.kernel(body,
                     out_type=jax.ShapeDtypeStruct((NC, nacc, width),
                                                   jnp.float32),
                     mesh=_sc_mesh(), scratch_types=scratch)


@functools.lru_cache(maxsize=None)
def _make_sc_cnt(nch, nacc):
    scratch = (
        pltpu.VMEM_SHARED((nacc, CW), jnp.float32),
        pltpu.VMEM((IB, CH), jnp.int32),
        pltpu.VMEM((CH, CW), jnp.float32),
        pltpu.VMEM((ZR, CW), jnp.float32),
    )
    body = functools.partial(_sc_cnt_body, nch, nacc)
    return pl.kernel(body,
                     out_type=jax.ShapeDtypeStruct((NC, nacc, CW),
                                                   jnp.float32),
                     mesh=_sc_mesh(), scratch_types=scratch)


_DN = (((1,), (1,)), ((), ()))


def _mm_in_body(x_ref, wl_ref, wr_ref, bl_ref, p_ref, r_ref):
    h = x_ref[...]
    p_ref[...] = lax.dot_general(h, wl_ref[...], _DN,
                                 preferred_element_type=jnp.float32)
    r_ref[...] = lax.dot_general(h, wr_ref[...], _DN,
                                 preferred_element_type=jnp.float32) + bl_ref[...]


def _combine_mm_body(s0_ref, s1_ref, r_ref, c0_ref, c1_ref, g_ref, be_ref,
                     wl_ref, wr_ref, bl_ref, p_ref, rn_ref):
    cnt = c0_ref[...][:, :1] + c1_ref[...][:, :1]
    rc = 1.0 / jnp.maximum(cnt, 1.0)
    h = (s0_ref[...] + s1_ref[...]) * rc + r_ref[...]
    scale = g_ref[...] * (1.0 / math.sqrt(1.0 + BN_EPS))
    h = jnp.maximum(h * scale + be_ref[...], 0.0)
    p_ref[...] = lax.dot_general(h, wl_ref[...], _DN,
                                 preferred_element_type=jnp.float32)
    rn_ref[...] = lax.dot_general(h, wr_ref[...], _DN,
                                  preferred_element_type=jnp.float32) + bl_ref[...]


def _final_body(s0_ref, s1_ref, r_ref, c0_ref, c1_ref, o_ref):
    cnt = c0_ref[...][:, :1] + c1_ref[...][:, :1]
    rc = 1.0 / jnp.maximum(cnt, 1.0)
    r = r_ref[...]
    agg = (s0_ref[...] + s1_ref[...]) * rc
    o_ref[...] = agg[:, :r.shape[1]] + r


def _row_spec(nb, w):
    return pl.BlockSpec((nb, w), lambda i: (i, 0))


def _full_spec(shape):
    return pl.BlockSpec(shape, lambda i: tuple(0 for _ in shape))


def kernel(x, edge_index, Wl0, bl0, Wr0, Wl1, bl1, Wr1, Wl2, bl2, Wr2,
           g0, be0, g1, be1):
    n, d = x.shape
    e = edge_index.shape[1]
    h_dim = Wl0.shape[0]
    o_dim = Wl2.shape[0]

    nch = -(-e // (NC * NS * CH))
    epad = NC * NS * CH * nch
    nacc = -(-(n + 1) // (NS * ZR)) * (NS * ZR)

    src = edge_index[0]
    dst = edge_index[1]
    pad = epad - e
    pad_dst = n + jnp.arange(pad, dtype=jnp.int32) % jnp.int32(nacc - n)
    src_r = jnp.concatenate(
        [src, jnp.zeros((pad,), jnp.int32)]).reshape(NC * NS, nch, CH)
    dst_r = jnp.concatenate([dst, pad_dst]).reshape(NC * NS, nch, CH)

    nb = 1000
    grid = (n // nb,)

    bl0_2, bl1_2, bl2_2 = bl0[None], bl1[None], bl2[None]
    g0_2, be0_2 = g0[None], be0[None]
    g1_2, be1_2 = g1[None], be1[None]

    cp = _make_sc_cnt(nch, nacc)(dst_r)

    p0, r0 = pl.pallas_call(
        _mm_in_body,
        grid=grid,
        in_specs=[_row_spec(nb, d), _full_spec((h_dim, d)),
                  _full_spec((h_dim, d)), _full_spec((1, h_dim))],
        out_specs=[_row_spec(nb, h_dim), _row_spec(nb, h_dim)],
        out_shape=[jax.ShapeDtypeStruct((n, h_dim), jnp.float32),
                   jax.ShapeDtypeStruct((n, h_dim), jnp.float32)],
    )(x, Wl0, Wr0, bl0_2)

    s0p = _make_sc_scatter(h_dim, nch, nacc)(p0, src_r, dst_r)
    s00, s01 = s0p[0, :n], s0p[1, :n]
    c0, c1 = cp[0, :n], cp[1, :n]

    p1, r1 = pl.pallas_call(
        _combine_mm_body,
        grid=grid,
        in_specs=[_row_spec(nb, h_dim), _row_spec(nb, h_dim),
                  _row_spec(nb, h_dim), _row_spec(nb, CW), _row_spec(nb, CW),
                  _full_spec((1, h_dim)), _full_spec((1, h_dim)),
                  _full_spec((h_dim, h_dim)), _full_spec((h_dim, h_dim)),
                  _full_spec((1, h_dim))],
        out_specs=[_row_spec(nb, h_dim), _row_spec(nb, h_dim)],
        out_shape=[jax.ShapeDtypeStruct((n, h_dim), jnp.float32),
                   jax.ShapeDtypeStruct((n, h_dim), jnp.float32)],
    )(s00, s01, r0, c0, c1, g0_2, be0_2, Wl1, Wr1, bl1_2)

    s1p = _make_sc_scatter(h_dim, nch, nacc)(p1, src_r, dst_r)

    wl2p = jnp.concatenate(
        [Wl2, jnp.zeros((h_dim - o_dim, h_dim), jnp.float32)], axis=0)
    p2, r2 = pl.pallas_call(
        _combine_mm_body,
        grid=grid,
        in_specs=[_row_spec(nb, h_dim), _row_spec(nb, h_dim),
                  _row_spec(nb, h_dim), _row_spec(nb, CW), _row_spec(nb, CW),
                  _full_spec((1, h_dim)), _full_spec((1, h_dim)),
                  _full_spec((h_dim, h_dim)), _full_spec((o_dim, h_dim)),
                  _full_spec((1, o_dim))],
        out_specs=[_row_spec(nb, h_dim), _row_spec(nb, o_dim)],
        out_shape=[jax.ShapeDtypeStruct((n, h_dim), jnp.float32),
                   jax.ShapeDtypeStruct((n, o_dim), jnp.float32)],
    )(s1p[0, :n], s1p[1, :n], r1, c0, c1, g1_2, be1_2, wl2p, Wr2, bl2_2)

    s2p = _make_sc_scatter(h_dim, nch, nacc)(p2, src_r, dst_r)

    out = pl.pallas_call(
        _final_body,
        grid=grid,
        in_specs=[_row_spec(nb, h_dim), _row_spec(nb, h_dim),
                  _row_spec(nb, o_dim), _row_spec(nb, CW), _row_spec(nb, CW)],
        out_specs=_row_spec(nb, o_dim),
        out_shape=jax.ShapeDtypeStruct((n, o_dim), jnp.float32),
    )(s2p[0, :n], s2p[1, :n], r2, c0, c1)

    return out
